# baseline (speedup 1.0000x reference)
import jax
import jax.numpy as jnp
from jax.experimental import pallas as pl
from jax.experimental.pallas import tpu as pltpu

BATCH, N, K, D = 8, 2048, 256, 512
A_ROWS = BATCH * N
JSTEPS = 8
A_BLK = N // JSTEPS
SC_PER_STEP = K // JSTEPS


def _body(idx_sref, a_ref, x_ref, a_out_ref, newx_ref):
    b = pl.program_id(0)
    j = pl.program_id(1)

    @pl.when(j == 0)
    def _():
        newx_ref[...] = jnp.zeros((N, D), jnp.float32)

    base = b * K + j * SC_PER_STEP
    for t in range(SC_PER_STEP):
        row = idx_sref[base + t]
        newx_ref[pl.ds(row, 1), :] = x_ref[pl.ds(j * SC_PER_STEP + t, 1), :]

    a_out_ref[...] = a_ref[...]


_fused = pl.pallas_call(
    _body,
    grid_spec=pltpu.PrefetchScalarGridSpec(
        num_scalar_prefetch=1,
        grid=(BATCH, JSTEPS),
        in_specs=[
            pl.BlockSpec((A_BLK, N), lambda b, j, idx: (b * JSTEPS + j, 0)),
            pl.BlockSpec((K, D), lambda b, j, idx: (b, 0)),
        ],
        out_specs=[
            pl.BlockSpec((A_BLK, N), lambda b, j, idx: (b * JSTEPS + j, 0)),
            pl.BlockSpec((N, D), lambda b, j, idx: (b, 0)),
        ],
    ),
    out_shape=(
        jax.ShapeDtypeStruct((A_ROWS, N), jnp.float32),
        jax.ShapeDtypeStruct((BATCH * N, D), jnp.float32),
    ),
    compiler_params=pltpu.CompilerParams(
        dimension_semantics=("arbitrary", "arbitrary"),
    ),
)


def kernel(A, X, idx_batch):
    a_flat = A.reshape(A_ROWS, N)
    x_flat = X.reshape(BATCH * K, D)
    idx_flat = idx_batch.reshape(BATCH * K).astype(jnp.int32)
    a_out, newx = _fused(idx_flat, a_flat, x_flat)
    return (a_out.reshape(BATCH, N, N), newx.reshape(BATCH, N, D))


# speedup vs baseline: 1.1016x; 1.1016x over previous
import jax
import jax.numpy as jnp
from jax.experimental import pallas as pl
from jax.experimental.pallas import tpu as pltpu

BATCH, N, K, D = 8, 2048, 256, 512
A_ROWS = BATCH * N
JSTEPS = 2
A_BLK = N // JSTEPS
SC_PER_STEP = K // JSTEPS


def _body(idx_sref, a_ref, x_ref, a_out_ref, newx_ref):
    b = pl.program_id(0)
    j = pl.program_id(1)

    @pl.when(j == 0)
    def _():
        newx_ref[...] = jnp.zeros((N, D), jnp.float32)

    base = b * K + j * SC_PER_STEP
    for t in range(SC_PER_STEP):
        row = idx_sref[base + t]
        newx_ref[pl.ds(row, 1), :] = x_ref[pl.ds(j * SC_PER_STEP + t, 1), :]

    a_out_ref[...] = a_ref[...]


_fused = pl.pallas_call(
    _body,
    grid_spec=pltpu.PrefetchScalarGridSpec(
        num_scalar_prefetch=1,
        grid=(BATCH, JSTEPS),
        in_specs=[
            pl.BlockSpec((A_BLK, N), lambda b, j, idx: (b * JSTEPS + j, 0)),
            pl.BlockSpec((K, D), lambda b, j, idx: (b, 0)),
        ],
        out_specs=[
            pl.BlockSpec((A_BLK, N), lambda b, j, idx: (b * JSTEPS + j, 0)),
            pl.BlockSpec((N, D), lambda b, j, idx: (b, 0)),
        ],
    ),
    out_shape=(
        jax.ShapeDtypeStruct((A_ROWS, N), jnp.float32),
        jax.ShapeDtypeStruct((BATCH * N, D), jnp.float32),
    ),
    compiler_params=pltpu.CompilerParams(
        dimension_semantics=("arbitrary", "arbitrary"),
    ),
)


def kernel(A, X, idx_batch):
    a_flat = A.reshape(A_ROWS, N)
    x_flat = X.reshape(BATCH * K, D)
    idx_flat = idx_batch.reshape(BATCH * K).astype(jnp.int32)
    a_out, newx = _fused(idx_flat, a_flat, x_flat)
    return (a_out.reshape(BATCH, N, N), newx.reshape(BATCH, N, D))
